# baseline (speedup 1.0000x reference)
import jax
import jax.numpy as jnp
from jax import lax
from jax.experimental import pallas as pl
from jax.experimental.pallas import tpu as pltpu
from jax.experimental.pallas import tpu_sc as plsc

_NUM_HEADS = 16
_EMBED = 3969
_S = 1024
_NC = 2
_NS = 16
_L = 16
_HPC = _NUM_HEADS // _NC
_ROWS_PER_W = _S // _NS
_SLAB_ROWS = 4
_SLABS = _ROWS_PER_W // _SLAB_ROWS
_SLAB = _SLAB_ROWS * _S
_GROUPS = _SLAB // _L


def _gather_body(table_hbm, idx_hbm, out_hbm,
                 table_v, idx0_v, idx1_v, out0_v, out1_v,
                 isem0, isem1, osem0, osem1):
  hh = lax.axis_index("c")
  rb = lax.axis_index("s")
  base = rb * (_ROWS_PER_W * _S)
  pltpu.sync_copy(table_hbm.at[pl.ds(hh * (_HPC * _EMBED), _HPC * _EMBED)],
                  table_v)

  ibufs = (idx0_v, idx1_v)
  obufs = (out0_v, out1_v)
  isems = (isem0, isem1)
  osems = (osem0, osem1)

  idx_copies = [
      pltpu.async_copy(idx_hbm.at[pl.ds(base, _SLAB)], idx0_v, isem0),
      pltpu.async_copy(idx_hbm.at[pl.ds(base + _SLAB, _SLAB)], idx1_v, isem1),
  ]
  out_copies = [None, None]

  for s in range(_SLABS):
    b = s % 2
    ibuf, obuf = ibufs[b], obufs[b]
    idx_copies[b].wait()
    if out_copies[b] is not None:
      for c in out_copies[b]:
        c.wait()

    def do_group(g, ibuf=ibuf, obuf=obuf):
      vi = ibuf[pl.ds(g * _L, _L)]
      for h in range(_HPC):
        obuf[h, pl.ds(g * _L, _L)] = plsc.load_gather(
            table_v, [vi + h * _EMBED])

    plsc.parallel_loop(0, _GROUPS, unroll=2)(do_group)

    off = base + s * _SLAB
    out_copies[b] = [
        pltpu.async_copy(obuf.at[h],
                         out_hbm.at[hh * _HPC + h, pl.ds(off, _SLAB)],
                         osems[b])
        for h in range(_HPC)
    ]
    if s + 2 < _SLABS:
      idx_copies[b] = pltpu.async_copy(
          idx_hbm.at[pl.ds(base + (s + 2) * _SLAB, _SLAB)], ibuf, isems[b])

  for b in range(2):
    for c in out_copies[b]:
      c.wait()


@jax.jit
def kernel(attn_rpe_index, relative_position_bias_table):
  idx_flat = attn_rpe_index.astype(jnp.int32).reshape(-1)
  table_flat = relative_position_bias_table.reshape(-1)
  mesh = plsc.VectorSubcoreMesh(
      core_axis_name="c", subcore_axis_name="s",
      num_cores=_NC, num_subcores=_NS)
  out = pl.kernel(
      _gather_body,
      out_type=jax.ShapeDtypeStruct((_NUM_HEADS, _S * _S), jnp.float32),
      mesh=mesh,
      scratch_types=[
          pltpu.VMEM((_HPC * _EMBED,), jnp.float32),
          pltpu.VMEM((_SLAB,), jnp.int32),
          pltpu.VMEM((_SLAB,), jnp.int32),
          pltpu.VMEM((_HPC, _SLAB), jnp.float32),
          pltpu.VMEM((_HPC, _SLAB), jnp.float32),
          pltpu.SemaphoreType.DMA,
          pltpu.SemaphoreType.DMA,
          pltpu.SemaphoreType.DMA,
          pltpu.SemaphoreType.DMA,
      ],
      compiler_params=pltpu.CompilerParams(needs_layout_passes=False),
  )(table_flat, idx_flat)
  return out.reshape(1, _NUM_HEADS, _S, _S)


# speedup vs baseline: 1.0317x; 1.0317x over previous
import jax
import jax.numpy as jnp
from jax import lax
from jax.experimental import pallas as pl
from jax.experimental.pallas import tpu as pltpu
from jax.experimental.pallas import tpu_sc as plsc

_NUM_HEADS = 16
_EMBED = 3969
_S = 1024
_NC = 2
_NS = 16
_L = 16
_HPC = _NUM_HEADS // _NC
_ROWS_PER_W = _S // _NS
_SLAB_ROWS = 4
_SLABS = _ROWS_PER_W // _SLAB_ROWS
_SLAB = _SLAB_ROWS * _S
_GROUPS = _SLAB // _L


def _gather_body(table_hbm, idx_hbm, out_hbm,
                 table_v, idx0_v, idx1_v, out0_v, out1_v,
                 isem0, isem1, osem0, osem1):
  hh = lax.axis_index("c")
  rb = lax.axis_index("s")
  base = rb * (_ROWS_PER_W * _S)
  pltpu.sync_copy(table_hbm.at[pl.ds(hh * (_HPC * _EMBED), _HPC * _EMBED)],
                  table_v)

  ibufs = (idx0_v, idx1_v)
  obufs = (out0_v, out1_v)
  isems = (isem0, isem1)
  osems = (osem0, osem1)

  pltpu.async_copy(idx_hbm.at[pl.ds(base, _SLAB)], idx0_v, isem0)
  pltpu.async_copy(idx_hbm.at[pl.ds(base + _SLAB, _SLAB)], idx1_v, isem1)

  def do_pair(k, carry):
    for b in range(2):
      s = 2 * k + b
      ibuf, obuf = ibufs[b], obufs[b]
      pltpu.make_async_copy(
          idx_hbm.at[pl.ds(base, _SLAB)], ibuf, isems[b]).wait()

      @pl.when(k > 0)
      def _wait(obuf=obuf, osem=osems[b]):
        for h in range(_HPC):
          pltpu.make_async_copy(
              obuf.at[h], out_hbm.at[0, pl.ds(0, _SLAB)], osem).wait()

      def do_group(g, ibuf=ibuf, obuf=obuf):
        vi = ibuf[pl.ds(g * _L, _L)]
        for h in range(_HPC):
          obuf[h, pl.ds(g * _L, _L)] = plsc.load_gather(
              table_v, [vi + h * _EMBED])

      plsc.parallel_loop(0, _GROUPS, unroll=2)(do_group)

      off = base + s * _SLAB
      for h in range(_HPC):
        pltpu.async_copy(obuf.at[h],
                         out_hbm.at[hh * _HPC + h, pl.ds(off, _SLAB)],
                         osems[b])

      @pl.when(s + 2 < _SLABS)
      def _prefetch(ibuf=ibuf, isem=isems[b], s=s):
        pltpu.async_copy(
            idx_hbm.at[pl.ds(base + (s + 2) * _SLAB, _SLAB)], ibuf, isem)
    return carry

  lax.fori_loop(0, _SLABS // 2, do_pair, 0)
  for b in range(2):
    for h in range(_HPC):
      pltpu.make_async_copy(
          obufs[b].at[h], out_hbm.at[0, pl.ds(0, _SLAB)], osems[b]).wait()


@jax.jit
def kernel(attn_rpe_index, relative_position_bias_table):
  idx_flat = attn_rpe_index.astype(jnp.int32).reshape(-1)
  table_flat = relative_position_bias_table.reshape(-1)
  mesh = plsc.VectorSubcoreMesh(
      core_axis_name="c", subcore_axis_name="s",
      num_cores=_NC, num_subcores=_NS)
  out = pl.kernel(
      _gather_body,
      out_type=jax.ShapeDtypeStruct((_NUM_HEADS, _S * _S), jnp.float32),
      mesh=mesh,
      scratch_types=[
          pltpu.VMEM((_HPC * _EMBED,), jnp.float32),
          pltpu.VMEM((_SLAB,), jnp.int32),
          pltpu.VMEM((_SLAB,), jnp.int32),
          pltpu.VMEM((_HPC, _SLAB), jnp.float32),
          pltpu.VMEM((_HPC, _SLAB), jnp.float32),
          pltpu.SemaphoreType.DMA,
          pltpu.SemaphoreType.DMA,
          pltpu.SemaphoreType.DMA,
          pltpu.SemaphoreType.DMA,
      ],
      compiler_params=pltpu.CompilerParams(needs_layout_passes=False),
  )(table_flat, idx_flat)
  return out.reshape(1, _NUM_HEADS, _S, _S)


# speedup vs baseline: 2.3731x; 2.3002x over previous
import jax
import jax.numpy as jnp
from jax import lax
from jax.experimental import pallas as pl
from jax.experimental.pallas import tpu as pltpu
from jax.experimental.pallas import tpu_sc as plsc

_NUM_HEADS = 16
_EMBED = 3969
_S = 1024
_NC = 2
_NS = 16
_L = 16
_HPC = _NUM_HEADS // _NC
_ROWS_PER_W = _S // _NS
_SLAB_ROWS = 4
_SLABS = _ROWS_PER_W // _SLAB_ROWS
_GPR = _S // _L
_GROUPS = _SLAB_ROWS * _GPR


def _gather_body(table_hbm, idx_hbm, out_hbm,
                 table_v, idx0_v, idx1_v, out0_v, out1_v,
                 isem0, isem1, osem0, osem1):
  hh = lax.axis_index("c")
  rb = lax.axis_index("s")
  row_base = rb * _ROWS_PER_W
  pltpu.sync_copy(table_hbm.at[pl.ds(hh * (_HPC * _EMBED), _HPC * _EMBED)],
                  table_v)

  ibufs = (idx0_v, idx1_v)
  obufs = (out0_v, out1_v)
  isems = (isem0, isem1)
  osems = (osem0, osem1)

  pltpu.async_copy(idx_hbm.at[pl.ds(row_base, _SLAB_ROWS)], idx0_v, isem0)
  pltpu.async_copy(idx_hbm.at[pl.ds(row_base + _SLAB_ROWS, _SLAB_ROWS)],
                   idx1_v, isem1)

  def do_pair(k, carry):
    for b in range(2):
      s = 2 * k + b
      ibuf, obuf = ibufs[b], obufs[b]
      pltpu.make_async_copy(
          idx_hbm.at[pl.ds(row_base, _SLAB_ROWS)], ibuf, isems[b]).wait()

      @pl.when(k > 0)
      def _wait(obuf=obuf, osem=osems[b]):
        for h in range(_HPC):
          pltpu.make_async_copy(
              obuf.at[h], out_hbm.at[0, pl.ds(0, _SLAB_ROWS)], osem).wait()

      def do_group(g, ibuf=ibuf, obuf=obuf):
        rr = g // _GPR
        gg = g - rr * _GPR
        vi = ibuf[rr, pl.ds(gg * _L, _L)]
        for h in range(_HPC):
          obuf[h, rr, pl.ds(gg * _L, _L)] = plsc.load_gather(
              table_v, [vi + h * _EMBED])

      plsc.parallel_loop(0, _GROUPS, unroll=2)(do_group)

      row = row_base + s * _SLAB_ROWS
      for h in range(_HPC):
        pltpu.async_copy(obuf.at[h],
                         out_hbm.at[hh * _HPC + h, pl.ds(row, _SLAB_ROWS)],
                         osems[b])

      @pl.when(s + 2 < _SLABS)
      def _prefetch(ibuf=ibuf, isem=isems[b], s=s):
        pltpu.async_copy(
            idx_hbm.at[pl.ds(row_base + (s + 2) * _SLAB_ROWS, _SLAB_ROWS)],
            ibuf, isem)
    return carry

  lax.fori_loop(0, _SLABS // 2, do_pair, 0)
  for b in range(2):
    for h in range(_HPC):
      pltpu.make_async_copy(
          obufs[b].at[h], out_hbm.at[0, pl.ds(0, _SLAB_ROWS)], osems[b]).wait()


@jax.jit
def kernel(attn_rpe_index, relative_position_bias_table):
  idx = attn_rpe_index.astype(jnp.int32)
  table_flat = relative_position_bias_table.reshape(-1)
  mesh = plsc.VectorSubcoreMesh(
      core_axis_name="c", subcore_axis_name="s",
      num_cores=_NC, num_subcores=_NS)
  out = pl.kernel(
      _gather_body,
      out_type=jax.ShapeDtypeStruct((_NUM_HEADS, _S, _S), jnp.float32),
      mesh=mesh,
      scratch_types=[
          pltpu.VMEM((_HPC * _EMBED,), jnp.float32),
          pltpu.VMEM((_SLAB_ROWS, _S), jnp.int32),
          pltpu.VMEM((_SLAB_ROWS, _S), jnp.int32),
          pltpu.VMEM((_HPC, _SLAB_ROWS, _S), jnp.float32),
          pltpu.VMEM((_HPC, _SLAB_ROWS, _S), jnp.float32),
          pltpu.SemaphoreType.DMA,
          pltpu.SemaphoreType.DMA,
          pltpu.SemaphoreType.DMA,
          pltpu.SemaphoreType.DMA,
      ],
      compiler_params=pltpu.CompilerParams(needs_layout_passes=False),
  )(table_flat, idx)
  return out[None]


# speedup vs baseline: 2.4086x; 1.0150x over previous
import jax
import jax.numpy as jnp
from jax import lax
from jax.experimental import pallas as pl
from jax.experimental.pallas import tpu as pltpu
from jax.experimental.pallas import tpu_sc as plsc

_NUM_HEADS = 16
_EMBED = 3969
_S = 1024
_NC = 2
_NS = 16
_L = 16
_HPC = _NUM_HEADS // _NC
_ROWS_PER_W = _S // _NS
_SLAB_ROWS = 4
_SLABS = _ROWS_PER_W // _SLAB_ROWS
_GPR = _S // _L
_GROUPS = _SLAB_ROWS * _GPR


def _gather_body(table_hbm, idx_hbm, out_hbm,
                 table_v, idx0_v, idx1_v, out0_v, out1_v,
                 isem0, isem1, osem0, osem1):
  hh = lax.axis_index("c")
  rb = lax.axis_index("s")
  row_base = rb * _ROWS_PER_W

  ibufs = (idx0_v, idx1_v)
  obufs = (out0_v, out1_v)
  isems = (isem0, isem1)
  osems = (osem0, osem1)

  pltpu.async_copy(idx_hbm.at[pl.ds(row_base, _SLAB_ROWS)], idx0_v, isem0)
  pltpu.async_copy(idx_hbm.at[pl.ds(row_base + _SLAB_ROWS, _SLAB_ROWS)],
                   idx1_v, isem1)
  pltpu.sync_copy(table_hbm.at[pl.ds(hh * (_HPC * _EMBED), _HPC * _EMBED)],
                  table_v)

  def do_pair(k, carry):
    for b in range(2):
      s = 2 * k + b
      ibuf, obuf = ibufs[b], obufs[b]
      pltpu.make_async_copy(
          idx_hbm.at[pl.ds(row_base, _SLAB_ROWS)], ibuf, isems[b]).wait()

      @pl.when(k > 0)
      def _wait(obuf=obuf, osem=osems[b]):
        for h in range(_HPC):
          pltpu.make_async_copy(
              obuf.at[h], out_hbm.at[0, pl.ds(0, _SLAB_ROWS)], osem).wait()

      def do_group(g, ibuf=ibuf, obuf=obuf):
        rr = g // _GPR
        gg = g - rr * _GPR
        vi = ibuf[rr, pl.ds(gg * _L, _L)]
        for h in range(_HPC):
          obuf[h, rr, pl.ds(gg * _L, _L)] = plsc.load_gather(
              table_v, [vi + h * _EMBED])

      plsc.parallel_loop(0, _GROUPS, unroll=2)(do_group)

      row = row_base + s * _SLAB_ROWS
      for h in range(_HPC):
        pltpu.async_copy(obuf.at[h],
                         out_hbm.at[hh * _HPC + h, pl.ds(row, _SLAB_ROWS)],
                         osems[b])

      @pl.when(s + 2 < _SLABS)
      def _prefetch(ibuf=ibuf, isem=isems[b], s=s):
        pltpu.async_copy(
            idx_hbm.at[pl.ds(row_base + (s + 2) * _SLAB_ROWS, _SLAB_ROWS)],
            ibuf, isem)
    return carry

  lax.fori_loop(0, _SLABS // 2, do_pair, 0)
  for b in range(2):
    for h in range(_HPC):
      pltpu.make_async_copy(
          obufs[b].at[h], out_hbm.at[0, pl.ds(0, _SLAB_ROWS)], osems[b]).wait()


@jax.jit
def kernel(attn_rpe_index, relative_position_bias_table):
  idx = attn_rpe_index.astype(jnp.int32)
  table_flat = relative_position_bias_table.reshape(-1)
  mesh = plsc.VectorSubcoreMesh(
      core_axis_name="c", subcore_axis_name="s",
      num_cores=_NC, num_subcores=_NS)
  out = pl.kernel(
      _gather_body,
      out_type=jax.ShapeDtypeStruct((_NUM_HEADS, _S, _S), jnp.float32),
      mesh=mesh,
      scratch_types=[
          pltpu.VMEM((_HPC * _EMBED,), jnp.float32),
          pltpu.VMEM((_SLAB_ROWS, _S), jnp.int32),
          pltpu.VMEM((_SLAB_ROWS, _S), jnp.int32),
          pltpu.VMEM((_HPC, _SLAB_ROWS, _S), jnp.float32),
          pltpu.VMEM((_HPC, _SLAB_ROWS, _S), jnp.float32),
          pltpu.SemaphoreType.DMA,
          pltpu.SemaphoreType.DMA,
          pltpu.SemaphoreType.DMA,
          pltpu.SemaphoreType.DMA,
      ],
      compiler_params=pltpu.CompilerParams(needs_layout_passes=False),
  )(table_flat, idx)
  return out[None]
